# baseline (speedup 1.0000x reference)
import functools
import math

import jax
import jax.numpy as jnp
from jax import lax
from jax.experimental import pallas as pl
from jax.experimental.pallas import tpu as pltpu
from jax.experimental.pallas import tpu_sc as plsc

SMOOTH = 0.1
VOCAB = 32000
FILL = SMOOTH / (VOCAB - 1)
CONF = 1.0 - SMOOTH
C_CONST = SMOOTH * math.log(FILL) + CONF * math.log(CONF)

ROWS_PER_BLOCK = 128
N_WORKERS = 32
LANES = 16


def _rowsum_kernel(pred_lo_ref, pred_hi_ref, tgt_ref, s_ref, cnt_ref,
                   acc_ref, acccnt_ref, *, num_blocks):
    i = pl.program_id(0)

    tgt = tgt_ref[...]
    rowsum = (jnp.sum(pred_lo_ref[...], axis=1, keepdims=True)
              + jnp.sum(pred_hi_ref[...], axis=1, keepdims=True))
    mask = (tgt != 0).astype(jnp.float32)
    block_loss = jnp.sum(mask * (C_CONST - FILL * rowsum))
    block_cnt = jnp.sum(mask)

    @pl.when(i == 0)
    def _():
        acc_ref[0, 0] = 0.0
        acccnt_ref[0, 0] = 0.0

    acc_ref[0, 0] += block_loss
    acccnt_ref[0, 0] += block_cnt

    @pl.when(i == num_blocks - 1)
    def _():
        s_ref[...] = jnp.full((1, 1), acc_ref[0, 0], dtype=jnp.float32)
        cnt_ref[...] = jnp.full((1, 1), acccnt_ref[0, 0], dtype=jnp.float32)


def _sc_gather(pred_hbm, tgt_hbm, out_hbm, tgt_v, idx_v, g_v, acc_v, sem):
    wid = lax.axis_index("s") * 2 + lax.axis_index("c")
    per_w = 2048 // N_WORKERS
    base = wid * per_w
    pltpu.sync_copy(tgt_hbm.at[pl.ds(base, per_w)], tgt_v)
    for j in range(per_w // LANES):
        sl = pl.ds(j * LANES, LANES)
        row = lax.iota(jnp.int32, LANES) + (base + j * LANES)
        idx_v[sl] = tgt_v[sl] + row * VOCAB
    pltpu.async_copy(pred_hbm.at[idx_v], g_v, sem).wait()
    acc = jnp.zeros((LANES,), jnp.float32)
    for j in range(per_w // LANES):
        sl = pl.ds(j * LANES, LANES)
        acc = acc + jnp.where(tgt_v[sl] != 0, g_v[sl], 0.0)
    acc_v[...] = acc
    pltpu.sync_copy(acc_v, out_hbm.at[wid])


def _combine_kernel(s_ref, cnt_ref, gp_ref, out_ref):
    gsum = jnp.sum(gp_ref[...])
    loss = (s_ref[0, 0] - (CONF - FILL) * gsum) / cnt_ref[0, 0]
    out_ref[...] = jnp.full((1, 1), loss, dtype=jnp.float32)


def kernel(prediction, target):
    _, n_tok, vocab = prediction.shape
    pred2d = prediction.reshape(n_tok, vocab)
    pred_flat = prediction.reshape(n_tok * vocab)
    tgt_col = target.reshape(n_tok, 1)
    tgt_flat = target.reshape(n_tok)
    num_blocks = n_tok // ROWS_PER_BLOCK
    half = vocab // 2

    s_part, cnt = pl.pallas_call(
        functools.partial(_rowsum_kernel, num_blocks=num_blocks),
        grid=(num_blocks,),
        in_specs=[
            pl.BlockSpec((ROWS_PER_BLOCK, half), lambda i: (i, 0)),
            pl.BlockSpec((ROWS_PER_BLOCK, half), lambda i: (i, 1)),
            pl.BlockSpec((ROWS_PER_BLOCK, 1), lambda i: (i, 0)),
        ],
        out_specs=[
            pl.BlockSpec((1, 1), lambda i: (0, 0)),
            pl.BlockSpec((1, 1), lambda i: (0, 0)),
        ],
        out_shape=[
            jax.ShapeDtypeStruct((1, 1), jnp.float32),
            jax.ShapeDtypeStruct((1, 1), jnp.float32),
        ],
        scratch_shapes=[
            pltpu.SMEM((1, 1), jnp.float32),
            pltpu.SMEM((1, 1), jnp.float32),
        ],
    )(pred2d, pred2d, tgt_col)

    per_w = n_tok // N_WORKERS
    sc_call = pl.kernel(
        _sc_gather,
        out_type=jax.ShapeDtypeStruct((N_WORKERS, LANES), jnp.float32),
        mesh=plsc.VectorSubcoreMesh(core_axis_name="c", subcore_axis_name="s"),
        scratch_types=[
            pltpu.VMEM((per_w,), jnp.int32),
            pltpu.VMEM((per_w,), jnp.int32),
            pltpu.VMEM((per_w,), jnp.float32),
            pltpu.VMEM((LANES,), jnp.float32),
            pltpu.SemaphoreType.DMA,
        ],
    )
    g_part = sc_call(pred_flat, tgt_flat)

    out = pl.pallas_call(
        _combine_kernel,
        grid=(1,),
        in_specs=[
            pl.BlockSpec((1, 1), lambda i: (0, 0)),
            pl.BlockSpec((1, 1), lambda i: (0, 0)),
            pl.BlockSpec((N_WORKERS, LANES), lambda i: (0, 0)),
        ],
        out_specs=pl.BlockSpec((1, 1), lambda i: (0, 0)),
        out_shape=jax.ShapeDtypeStruct((1, 1), jnp.float32),
    )(s_part, cnt, g_part)
    return out[0, 0]


# speedup vs baseline: 2.7863x; 2.7863x over previous
import functools
import math

import jax
import jax.numpy as jnp
from jax import lax
from jax.experimental import pallas as pl
from jax.experimental.pallas import tpu as pltpu
from jax.experimental.pallas import tpu_sc as plsc

SMOOTH = 0.1
VOCAB = 32000
FILL = SMOOTH / (VOCAB - 1)
CONF = 1.0 - SMOOTH
C_CONST = SMOOTH * math.log(FILL) + CONF * math.log(CONF)

ROWS_PER_BLOCK = 128
LANES = 16
ACC_LANES = 256


def _stream_kernel(pred_lo_ref, pred_hi_ref, tgt_ref, rm_ref, *,
                   num_blocks, half):
    i = pl.program_id(0)

    tgt = tgt_ref[...]
    lo = pred_lo_ref[...]
    hi = pred_hi_ref[...]
    col = jax.lax.broadcasted_iota(jnp.int32, lo.shape, 1)
    w_lo = jnp.where(col == tgt, CONF, FILL)
    w_hi = jnp.where(col + half == tgt, CONF, FILL)
    wsum = (jnp.sum(w_lo * lo, axis=1, keepdims=True)
            + jnp.sum(w_hi * hi, axis=1, keepdims=True))
    mask = (tgt != 0).astype(jnp.float32)
    block_loss = jnp.sum(mask * (C_CONST - wsum))
    block_cnt = jnp.sum(mask)

    lane = jax.lax.broadcasted_iota(jnp.int32, (1, ACC_LANES), 1)
    contrib = jnp.where(lane < 128, block_loss, block_cnt)

    @pl.when(i == 0)
    def _():
        rm_ref[...] = contrib

    @pl.when(i != 0)
    def _():
        rm_ref[...] += contrib


def _sc_combine(rm_hbm, out_hbm, rm_v, res_v):
    wid = lax.axis_index("s") * 2 + lax.axis_index("c")

    @pl.when(wid == 0)
    def _():
        pltpu.sync_copy(rm_hbm, rm_v)
        loss_tot = rm_v[pl.ds(0, LANES)]
        cnt_tot = rm_v[pl.ds(128, LANES)]
        res_v[...] = loss_tot / cnt_tot
        pltpu.sync_copy(res_v, out_hbm)


def kernel(prediction, target):
    _, n_tok, vocab = prediction.shape
    pred2d = prediction.reshape(n_tok, vocab)
    tgt_col = target.reshape(n_tok, 1)
    num_blocks = n_tok // ROWS_PER_BLOCK
    half = vocab // 2

    rm = pl.pallas_call(
        functools.partial(_stream_kernel, num_blocks=num_blocks, half=half),
        grid=(num_blocks,),
        in_specs=[
            pl.BlockSpec((ROWS_PER_BLOCK, half), lambda i: (i, 0)),
            pl.BlockSpec((ROWS_PER_BLOCK, half), lambda i: (i, 1)),
            pl.BlockSpec((ROWS_PER_BLOCK, 1), lambda i: (i, 0)),
        ],
        out_specs=pl.BlockSpec((1, ACC_LANES), lambda i: (0, 0)),
        out_shape=jax.ShapeDtypeStruct((1, ACC_LANES), jnp.float32),
    )(pred2d, pred2d, tgt_col)

    sc_call = pl.kernel(
        _sc_combine,
        out_type=jax.ShapeDtypeStruct((LANES,), jnp.float32),
        mesh=plsc.VectorSubcoreMesh(core_axis_name="c", subcore_axis_name="s"),
        scratch_types=[
            pltpu.VMEM((ACC_LANES,), jnp.float32),
            pltpu.VMEM((LANES,), jnp.float32),
        ],
    )
    out = sc_call(rm.reshape(ACC_LANES))
    return out[0]
